# Initial kernel scaffold; baseline (speedup 1.0000x reference)
import jax
import jax.numpy as jnp
from jax.experimental import pallas as pl


def kernel(batch, weights, locations, radius):
    raise NotImplementedError("write your pallas kernel here")


# speedup vs baseline: 6.6040x; 6.6040x over previous
import jax
import jax.numpy as jnp
from jax import lax
from jax.experimental import pallas as pl


def _som_kernel(batch_ref, w_ref, locT_ref, invr2_ref, out_ref):
    b = batch_ref[...]
    w = w_ref[...]
    locT = locT_ref[...]
    inv_r2 = invr2_ref[0, 0]

    bw = lax.dot_general(b, w, (((1,), (1,)), ((), ())),
                         preferred_element_type=jnp.float32,
                         precision=lax.Precision.HIGHEST)
    ones_row = jnp.ones((1, b.shape[1]), dtype=jnp.float32)
    wn = lax.dot_general(ones_row, w * w, (((1,), (1,)), ((), ())),
                         preferred_element_type=jnp.float32,
                         precision=lax.Precision.HIGHEST)
    scores = wn - 2.0 * bw

    row_min = jnp.min(scores, axis=1, keepdims=True)
    col = lax.broadcasted_iota(jnp.int32, scores.shape, 1)
    m_total = scores.shape[1]
    idx = jnp.min(jnp.where(scores <= row_min, col, m_total),
                  axis=1, keepdims=True)

    onehot = (col == idx)
    loc_i = locT[0:1, :]
    loc_j = locT[1:2, :]
    bi = jnp.sum(jnp.where(onehot, loc_i, 0.0), axis=1, keepdims=True)
    bj = jnp.sum(jnp.where(onehot, loc_j, 0.0), axis=1, keepdims=True)

    d2 = (loc_i - bi) ** 2 + (loc_j - bj) ** 2
    out_ref[...] = jnp.exp(-(d2 * inv_r2))


def kernel(batch, weights, locations, radius):
    locT = locations.astype(jnp.float32).T
    inv_r2 = (1.0 / (jnp.asarray(radius).astype(jnp.float32) ** 2)
              ).reshape(1, 1)
    B = batch.shape[0]
    M = weights.shape[0]
    return pl.pallas_call(
        _som_kernel,
        out_shape=jax.ShapeDtypeStruct((B, M), jnp.float32),
    )(batch, weights, locT, inv_r2)
